# baseline (speedup 1.0000x reference)
import functools

import jax
import jax.numpy as jnp
from jax import lax
from jax.experimental import pallas as pl
from jax.experimental.pallas import tpu as pltpu, tpu_sc as plsc

B = 1024
N = 2048
LANES = 16
NCHUNK = N // LANES
OUTWORDS = 8 * N

NC, NS = 2, 16
NW = NC * NS
BPW = B // NW

_mesh = plsc.VectorSubcoreMesh(
    core_axis_name="c", subcore_axis_name="s", num_cores=NC, num_subcores=NS
)


@functools.partial(
    pl.kernel,
    out_type=jax.ShapeDtypeStruct((B, 8, N), jnp.int32),
    mesh=_mesh,
    compiler_params=pltpu.CompilerParams(needs_layout_passes=False),
    scratch_types=[
        pltpu.VMEM((N,), jnp.float32),
        pltpu.VMEM((N,), jnp.float32),
        pltpu.VMEM((N,), jnp.float32),
        pltpu.VMEM((N,), jnp.float32),
        pltpu.VMEM((N,), jnp.float32),
        pltpu.VMEM((N,), jnp.float32),
        pltpu.VMEM((8, N), jnp.int32),
        pltpu.VMEM((8, N), jnp.int32),
        pltpu.VMEM((LANES,), jnp.int32),
        pltpu.VMEM((LANES,), jnp.int32),
        pltpu.VMEM((LANES,), jnp.int32),
        pltpu.SemaphoreType.DMA,
        pltpu.SemaphoreType.DMA,
        pltpu.SemaphoreType.DMA,
        pltpu.SemaphoreType.DMA,
    ],
)
def _octant_kernel(pcs_hbm, out_hbm, x0_v, y0_v, z0_v, x1_v, y1_v, z1_v,
                   out0_v, out1_v, cnt_v,
                   prev0_v, prev1_v, isem0, isem1, osem0, osem1):
    wid = lax.axis_index("s") * NC + lax.axis_index("c")
    b0 = wid * BPW
    zeros16 = jnp.zeros((LANES,), jnp.int32)
    iota16 = lax.iota(jnp.int32, LANES)
    xyzs = ((x0_v, y0_v, z0_v), (x1_v, y1_v, z1_v))
    outs = (out0_v, out1_v)
    isems = (isem0, isem1)
    osems = (osem0, osem1)

    for p in range(2):
        for c in range(3):
            pltpu.make_async_copy(
                pcs_hbm.at[c, b0 + p], xyzs[p][c], isems[p]
            ).start()

    def z0body(i, _):
        for p in range(2):
            for o in range(8):
                for j in range(2):
                    outs[p][o, pl.ds((2 * i + j) * LANES, LANES)] = zeros16
        return 0

    lax.fori_loop(0, N // (2 * LANES), z0body, 0)
    prev0_v[...] = zeros16
    prev1_v[...] = zeros16
    prevs = (prev0_v, prev1_v)

    def pair_body(t, _):
        for p in range(2):
            k = 2 * t + p
            b = b0 + k

            @pl.when(t > 0)
            def _():
                pltpu.make_async_copy(
                    outs[p], out_hbm.at[b - 2], osems[p]
                ).wait()

            cnt_v[...] = zeros16

            for c in range(3):
                pltpu.make_async_copy(
                    pcs_hbm.at[c, b], xyzs[p][c], isems[p]
                ).wait()

            def chunk_body(u, _):
                for v in range(8):
                    base = (NCHUNK - 1 - (8 * u + v)) * LANES
                    xv = xyzs[p][0][pl.ds(base, LANES)]
                    yv = xyzs[p][1][pl.ds(base, LANES)]
                    zv = xyzs[p][2][pl.ds(base, LANES)]
                    octant = (
                        jnp.where(xv > 0.0, jnp.int32(4), jnp.int32(0))
                        + jnp.where(yv > 0.0, jnp.int32(2), jnp.int32(0))
                        + jnp.where(zv > 0.0, jnp.int32(1), jnp.int32(0))
                    )
                    octr = lax.rev(octant, (0,))
                    idxr = (base + LANES - 1) - iota16
                    rank, last = plsc.scan_count(octr)
                    old = plsc.load_gather(cnt_v.at[:], [octr])
                    newcnt = old + rank
                    plsc.store_scatter(outs[p].at[:, :], [octr, newcnt - 1], idxr)
                    plsc.store_scatter(cnt_v.at[:], [octr], newcnt, mask=last)
                return 0

            lax.fori_loop(0, NCHUNK // 8, chunk_body, 0)

            tvec = cnt_v[...]
            pvec = prevs[p][...]
            diff = jnp.maximum(pvec - tvec, 0)
            maxd = jnp.max(diff)
            o8 = iota16 & 7
            tl = tvec.at[o8].get(mode="promise_in_bounds")
            pv = pvec.at[o8].get(mode="promise_in_bounds")
            jl = iota16 >> 3

            def zstep(g, _):
                idx = tl + 2 * g + jl
                m = idx < pv
                plsc.store_scatter(
                    outs[p].at[:, :],
                    [o8, jnp.minimum(idx, jnp.int32(N - 1))], zeros16, mask=m)
                return 0

            lax.fori_loop(0, (maxd + 1) >> 1, zstep, 0)
            prevs[p][...] = tvec

            pltpu.make_async_copy(
                outs[p], out_hbm.at[b], osems[p]
            ).start()

            @pl.when(k < BPW - 2)
            def _():
                for c in range(3):
                    pltpu.make_async_copy(
                        pcs_hbm.at[c, b + 2], xyzs[p][c], isems[p]
                    ).start()

        return 0

    lax.fori_loop(0, BPW // 2, pair_body, 0)

    for p in range(2):
        pltpu.make_async_copy(
            outs[p], out_hbm.at[b0 + BPW - 2 + p], osems[p]
        ).wait()


def kernel(pcs):
    return _octant_kernel(jnp.transpose(pcs, (1, 0, 2)))


# speedup vs baseline: 1.0026x; 1.0026x over previous
import functools

import jax
import jax.numpy as jnp
from jax import lax
from jax.experimental import pallas as pl
from jax.experimental.pallas import tpu as pltpu, tpu_sc as plsc

B = 1024
N = 2048
LANES = 16
NCHUNK = N // LANES
OUTWORDS = 8 * N

NC, NS = 2, 16
NW = NC * NS
BPW = B // NW

_mesh = plsc.VectorSubcoreMesh(
    core_axis_name="c", subcore_axis_name="s", num_cores=NC, num_subcores=NS
)


@functools.partial(
    pl.kernel,
    out_type=jax.ShapeDtypeStruct((B, 8, N), jnp.int32),
    mesh=_mesh,
    compiler_params=pltpu.CompilerParams(needs_layout_passes=False),
    scratch_types=[
        pltpu.VMEM((N,), jnp.float32),
        pltpu.VMEM((N,), jnp.float32),
        pltpu.VMEM((N,), jnp.float32),
        pltpu.VMEM((N,), jnp.float32),
        pltpu.VMEM((N,), jnp.float32),
        pltpu.VMEM((N,), jnp.float32),
        pltpu.VMEM((8, N), jnp.int32),
        pltpu.VMEM((8, N), jnp.int32),
        pltpu.VMEM((LANES,), jnp.int32),
        pltpu.VMEM((LANES,), jnp.int32),
        pltpu.VMEM((LANES,), jnp.int32),
        pltpu.SemaphoreType.DMA,
        pltpu.SemaphoreType.DMA,
        pltpu.SemaphoreType.DMA,
        pltpu.SemaphoreType.DMA,
    ],
)
def _octant_kernel(pcs_hbm, out_hbm, x0_v, y0_v, z0_v, x1_v, y1_v, z1_v,
                   out0_v, out1_v, cnt_v,
                   prev0_v, prev1_v, isem0, isem1, osem0, osem1):
    wid = lax.axis_index("s") * NC + lax.axis_index("c")
    b0 = wid * BPW
    zeros16 = jnp.zeros((LANES,), jnp.int32)
    iota16 = lax.iota(jnp.int32, LANES)
    xyzs = ((x0_v, y0_v, z0_v), (x1_v, y1_v, z1_v))
    outs = (out0_v, out1_v)
    isems = (isem0, isem1)
    osems = (osem0, osem1)

    for p in range(2):
        for c in range(3):
            pltpu.make_async_copy(
                pcs_hbm.at[c, b0 + p], xyzs[p][c], isems[p]
            ).start()

    def z0body(i, _):
        for p in range(2):
            for o in range(8):
                for j in range(2):
                    outs[p][o, pl.ds((2 * i + j) * LANES, LANES)] = zeros16
        return 0

    lax.fori_loop(0, N // (2 * LANES), z0body, 0)
    prev0_v[...] = zeros16
    prev1_v[...] = zeros16
    prevs = (prev0_v, prev1_v)

    def pair_body(t, _):
        for p in range(2):
            k = 2 * t + p
            b = b0 + k

            @pl.when(t > 0)
            def _():
                pltpu.make_async_copy(
                    outs[p], out_hbm.at[b - 2], osems[p]
                ).wait()

            cnt_v[...] = zeros16

            for c in range(3):
                pltpu.make_async_copy(
                    pcs_hbm.at[c, b], xyzs[p][c], isems[p]
                ).wait()

            def chunk_body(u, _):
                for v in range(4):
                    base = (NCHUNK - 1 - (4 * u + v)) * LANES
                    xv = xyzs[p][0][pl.ds(base, LANES)]
                    yv = xyzs[p][1][pl.ds(base, LANES)]
                    zv = xyzs[p][2][pl.ds(base, LANES)]
                    octant = (
                        jnp.where(xv > 0.0, jnp.int32(4), jnp.int32(0))
                        + jnp.where(yv > 0.0, jnp.int32(2), jnp.int32(0))
                        + jnp.where(zv > 0.0, jnp.int32(1), jnp.int32(0))
                    )
                    octr = lax.rev(octant, (0,))
                    idxr = (base + LANES - 1) - iota16
                    rank, last = plsc.scan_count(octr)
                    old = plsc.load_gather(cnt_v.at[:], [octr])
                    newcnt = old + rank
                    plsc.store_scatter(outs[p].at[:, :], [octr, newcnt - 1], idxr)
                    plsc.store_scatter(cnt_v.at[:], [octr], newcnt, mask=last)
                return 0

            lax.fori_loop(0, NCHUNK // 4, chunk_body, 0)

            tvec = cnt_v[...]
            pvec = prevs[p][...]
            diff = jnp.maximum(pvec - tvec, 0)
            maxd = jnp.max(diff)
            o8 = iota16 & 7
            tl = tvec.at[o8].get(mode="promise_in_bounds")
            pv = pvec.at[o8].get(mode="promise_in_bounds")
            jl = iota16 >> 3

            def zstep(g, _):
                idx = tl + 2 * g + jl
                m = idx < pv
                plsc.store_scatter(
                    outs[p].at[:, :],
                    [o8, jnp.minimum(idx, jnp.int32(N - 1))], zeros16, mask=m)
                return 0

            lax.fori_loop(0, (maxd + 1) >> 1, zstep, 0)
            prevs[p][...] = tvec

            pltpu.make_async_copy(
                outs[p], out_hbm.at[b], osems[p]
            ).start()

            @pl.when(k < BPW - 2)
            def _():
                for c in range(3):
                    pltpu.make_async_copy(
                        pcs_hbm.at[c, b + 2], xyzs[p][c], isems[p]
                    ).start()

        return 0

    lax.fori_loop(0, BPW // 2, pair_body, 0)

    for p in range(2):
        pltpu.make_async_copy(
            outs[p], out_hbm.at[b0 + BPW - 2 + p], osems[p]
        ).wait()


def kernel(pcs):
    return _octant_kernel(jnp.transpose(pcs, (1, 0, 2)))


# speedup vs baseline: 1.0036x; 1.0011x over previous
import functools

import jax
import jax.numpy as jnp
from jax import lax
from jax.experimental import pallas as pl
from jax.experimental.pallas import tpu as pltpu, tpu_sc as plsc

B = 1024
N = 2048
LANES = 16
NCHUNK = N // LANES

NC, NS = 2, 16
NW = NC * NS
BPW = B // NW

_mesh = plsc.VectorSubcoreMesh(
    core_axis_name="c", subcore_axis_name="s", num_cores=NC, num_subcores=NS
)


@functools.partial(
    pl.kernel,
    out_type=jax.ShapeDtypeStruct((B, 8, N), jnp.int32),
    mesh=_mesh,
    compiler_params=pltpu.CompilerParams(needs_layout_passes=False),
    scratch_types=[
        pltpu.VMEM((N,), jnp.float32),
        pltpu.VMEM((N,), jnp.float32),
        pltpu.VMEM((N,), jnp.float32),
        pltpu.VMEM((N,), jnp.float32),
        pltpu.VMEM((N,), jnp.float32),
        pltpu.VMEM((N,), jnp.float32),
        pltpu.VMEM((8, N), jnp.int32),
        pltpu.VMEM((8, N), jnp.int32),
        pltpu.VMEM((LANES,), jnp.int32),
        pltpu.VMEM((LANES,), jnp.int32),
        pltpu.VMEM((LANES,), jnp.int32),
        pltpu.SemaphoreType.DMA,
        pltpu.SemaphoreType.DMA,
        pltpu.SemaphoreType.DMA,
        pltpu.SemaphoreType.DMA,
    ],
)
def _octant_kernel(pcs_hbm, out_hbm, x0_v, y0_v, z0_v, x1_v, y1_v, z1_v,
                   out0_v, out1_v, cnt_v,
                   prev0_v, prev1_v, isem0, isem1, osem0, osem1):
    wid = lax.axis_index("s") * NC + lax.axis_index("c")
    b0 = wid * BPW
    zeros16 = jnp.zeros((LANES,), jnp.int32)
    iota16 = lax.iota(jnp.int32, LANES)
    xyzs = ((x0_v, y0_v, z0_v), (x1_v, y1_v, z1_v))
    outs = (out0_v, out1_v)
    isems = (isem0, isem1)
    osems = (osem0, osem1)

    for p in range(2):
        for c in range(3):
            pltpu.make_async_copy(
                pcs_hbm.at[c, b0 + p], xyzs[p][c], isems[p]
            ).start()

    def z0body(i, _):
        for p in range(2):
            for o in range(8):
                for j in range(2):
                    outs[p][o, pl.ds((2 * i + j) * LANES, LANES)] = zeros16
        return 0

    lax.fori_loop(0, N // (2 * LANES), z0body, 0)
    prev0_v[...] = zeros16
    prev1_v[...] = zeros16
    prevs = (prev0_v, prev1_v)

    def pair_body(t, _):
        for p in range(2):
            k = 2 * t + p
            b = b0 + k

            @pl.when(t > 0)
            def _():
                pltpu.make_async_copy(
                    outs[p], out_hbm.at[b - 2], osems[p]
                ).wait()

            cnt_v[...] = zeros16

            for c in range(3):
                pltpu.make_async_copy(
                    pcs_hbm.at[c, b], xyzs[p][c], isems[p]
                ).wait()

            def chunk_body(u, _):
                for v in range(4):
                    base = (NCHUNK - 1 - (4 * u + v)) * LANES
                    xv = xyzs[p][0][pl.ds(base, LANES)]
                    yv = xyzs[p][1][pl.ds(base, LANES)]
                    zv = xyzs[p][2][pl.ds(base, LANES)]
                    octant = (
                        jnp.where(xv > 0.0, jnp.int32(4), jnp.int32(0))
                        + jnp.where(yv > 0.0, jnp.int32(2), jnp.int32(0))
                        + jnp.where(zv > 0.0, jnp.int32(1), jnp.int32(0))
                    )
                    octr = lax.rev(octant, (0,))
                    idxr = (base + LANES - 1) - iota16
                    rank, last = plsc.scan_count(octr)
                    old = plsc.load_gather(cnt_v.at[:], [octr])
                    newcnt = old + rank
                    plsc.store_scatter(outs[p].at[:, :], [octr, newcnt - 1], idxr)
                    plsc.store_scatter(cnt_v.at[:], [octr], newcnt, mask=last)
                return 0

            lax.fori_loop(0, NCHUNK // 4, chunk_body, 0)

            tvec = cnt_v[...]
            pvec = prevs[p][...]
            diff = jnp.maximum(pvec - tvec, 0)
            maxd = jnp.max(diff)
            o8 = iota16 & 7
            tl = tvec.at[o8].get(mode="promise_in_bounds")
            pv = pvec.at[o8].get(mode="promise_in_bounds")
            jl = iota16 >> 3

            def zstep(g, _):
                idx = tl + 2 * g + jl
                m = idx < pv
                plsc.store_scatter(
                    outs[p].at[:, :],
                    [o8, jnp.minimum(idx, jnp.int32(N - 1))], zeros16, mask=m)
                return 0

            lax.fori_loop(0, (maxd + 1) >> 1, zstep, 0)
            prevs[p][...] = tvec

            pltpu.make_async_copy(
                outs[p], out_hbm.at[b], osems[p]
            ).start()

            @pl.when(k < BPW - 2)
            def _():
                for c in range(3):
                    pltpu.make_async_copy(
                        pcs_hbm.at[c, b + 2], xyzs[p][c], isems[p]
                    ).start()

        return 0

    lax.fori_loop(0, BPW // 2, pair_body, 0)

    for p in range(2):
        pltpu.make_async_copy(
            outs[p], out_hbm.at[b0 + BPW - 2 + p], osems[p]
        ).wait()


def kernel(pcs):
    return _octant_kernel(jnp.transpose(pcs, (1, 0, 2)))
